# Initial kernel scaffold; baseline (speedup 1.0000x reference)
import functools

import jax
import jax.numpy as jnp
from jax import lax
from jax.experimental import pallas as pl
from jax.experimental.pallas import tpu as pltpu, tpu_sc as plsc

B, L = 4096, 200
TOK_D = 64
POS_D = 64
N = B * L
C = 128
NC, NS = 2, 16
NW = NC * NS
S = N // (NW * C)
ROWS = N // C


def _sc_embed(tok2, pos2, token_table, pos_table):
    mesh = plsc.VectorSubcoreMesh(core_axis_name="c", subcore_axis_name="s")

    @functools.partial(
        pl.kernel,
        mesh=mesh,
        out_type=jax.ShapeDtypeStruct((N, TOK_D + POS_D), jnp.float32),
        scratch_types=[
            pltpu.VMEM((S, C), jnp.int32),
            pltpu.VMEM((S, C), jnp.int32),
            pltpu.VMEM((C, TOK_D), jnp.float32),
            pltpu.VMEM((C, POS_D), jnp.float32),
            pltpu.SemaphoreType.DMA,
            pltpu.SemaphoreType.DMA,
        ],
    )
    def k(tok_hbm, pos_hbm, ttab_hbm, ptab_hbm, out_hbm,
          tok_idx, pos_idx, tok_rows, pos_rows, sem_t, sem_p):
        wid = lax.axis_index("s") * NC + lax.axis_index("c")
        row0 = wid * S
        pltpu.sync_copy(tok_hbm.at[pl.ds(row0, S)], tok_idx)
        pltpu.sync_copy(pos_hbm.at[pl.ds(row0, S)], pos_idx)

        def step(j, carry):
            g = (row0 + j) * C
            pltpu.async_copy(ttab_hbm.at[tok_idx.at[j]], tok_rows, sem_t).wait()
            pltpu.async_copy(ptab_hbm.at[pos_idx.at[j]], pos_rows, sem_p).wait()
            pltpu.sync_copy(tok_rows, out_hbm.at[pl.ds(g, C), pl.ds(0, TOK_D)])
            pltpu.sync_copy(pos_rows, out_hbm.at[pl.ds(g, C), pl.ds(TOK_D, POS_D)])
            return carry

        lax.fori_loop(0, S, step, 0)

    return k(tok2, pos2, token_table, pos_table)


def kernel(tokens, pos, token_table, pos_table):
    tok2 = tokens.reshape(ROWS, C)
    pos2 = pos.reshape(ROWS, C)
    out = _sc_embed(tok2, pos2, token_table, pos_table)
    return out.reshape(B, L, TOK_D + POS_D)


# speedup vs baseline: 3.2315x; 3.2315x over previous
import functools

import jax
import jax.numpy as jnp
from jax import lax
from jax.experimental import pallas as pl
from jax.experimental.pallas import tpu as pltpu, tpu_sc as plsc

B, L = 4096, 200
TOK_D = 64
POS_D = 64
N = B * L
C = 128
NC, NS = 2, 16
NW = NC * NS
S = N // (NW * C)
ROWS = N // C


def _sc_embed(tok2, pos2, token_table, pos_table):
    mesh = plsc.VectorSubcoreMesh(core_axis_name="c", subcore_axis_name="s")

    @functools.partial(
        pl.kernel,
        mesh=mesh,
        out_type=jax.ShapeDtypeStruct((N, TOK_D + POS_D), jnp.float32),
        compiler_params=pltpu.CompilerParams(use_tc_tiling_on_sc=False),
        scratch_types=[
            pltpu.VMEM((S, C), jnp.int32),
            pltpu.VMEM((S, C), jnp.int32),
            pltpu.VMEM((C, TOK_D), jnp.float32),
            pltpu.VMEM((C, POS_D), jnp.float32),
            pltpu.SemaphoreType.DMA,
            pltpu.SemaphoreType.DMA,
        ],
    )
    def k(tok_hbm, pos_hbm, ttab_hbm, ptab_hbm, out_hbm,
          tok_idx, pos_idx, tok_rows, pos_rows, sem_t, sem_p):
        wid = lax.axis_index("s") * NC + lax.axis_index("c")
        row0 = wid * S
        pltpu.sync_copy(tok_hbm.at[pl.ds(row0, S)], tok_idx)
        pltpu.sync_copy(pos_hbm.at[pl.ds(row0, S)], pos_idx)

        def step(j, carry):
            g = (row0 + j) * C
            pltpu.async_copy(ttab_hbm.at[tok_idx.at[j]], tok_rows, sem_t).wait()
            pltpu.async_copy(ptab_hbm.at[pos_idx.at[j]], pos_rows, sem_p).wait()
            pltpu.sync_copy(tok_rows, out_hbm.at[pl.ds(g, C), pl.ds(0, TOK_D)])
            pltpu.sync_copy(pos_rows, out_hbm.at[pl.ds(g, C), pl.ds(TOK_D, POS_D)])
            return carry

        lax.fori_loop(0, S, step, 0)

    return k(tok2, pos2, token_table, pos_table)


def kernel(tokens, pos, token_table, pos_table):
    tok2 = tokens.reshape(ROWS, C)
    pos2 = pos.reshape(ROWS, C)
    out = _sc_embed(tok2, pos2, token_table, pos_table)
    return out.reshape(B, L, TOK_D + POS_D)


# speedup vs baseline: 4.0847x; 1.2640x over previous
import functools

import jax
import jax.numpy as jnp
from jax import lax
from jax.experimental import pallas as pl
from jax.experimental.pallas import tpu as pltpu, tpu_sc as plsc

B, L = 4096, 200
TOK_D = 64
POS_D = 64
N = B * L
C = 128
NC, NS = 2, 16
NW = NC * NS
S = N // (NW * C)
ROWS = N // C
NBUF = 4


def _sc_embed(tok2, pos2, token_table, pos_table):
    mesh = plsc.VectorSubcoreMesh(core_axis_name="c", subcore_axis_name="s")

    scratch = (
        [pltpu.VMEM((S, C), jnp.int32)] * 2
        + [pltpu.VMEM((C, TOK_D), jnp.float32)] * NBUF
        + [pltpu.VMEM((C, POS_D), jnp.float32)] * NBUF
        + [pltpu.SemaphoreType.DMA] * (2 * NBUF)
    )

    @functools.partial(
        pl.kernel,
        mesh=mesh,
        out_type=jax.ShapeDtypeStruct((N, TOK_D + POS_D), jnp.float32),
        compiler_params=pltpu.CompilerParams(use_tc_tiling_on_sc=False),
        scratch_types=scratch,
    )
    def k(tok_hbm, pos_hbm, ttab_hbm, ptab_hbm, out_hbm, tok_idx, pos_idx,
          *bufs):
        tok_rows = bufs[0:NBUF]
        pos_rows = bufs[NBUF:2 * NBUF]
        sem_g = bufs[2 * NBUF:3 * NBUF]
        sem_w = bufs[3 * NBUF:4 * NBUF]

        wid = lax.axis_index("s") * NC + lax.axis_index("c")
        row0 = wid * S
        pltpu.sync_copy(tok_hbm.at[pl.ds(row0, S)], tok_idx)
        pltpu.sync_copy(pos_hbm.at[pl.ds(row0, S)], pos_idx)

        def issue_gather(j, b):
            pltpu.async_copy(ttab_hbm.at[tok_idx.at[j]], tok_rows[b], sem_g[b])
            pltpu.async_copy(ptab_hbm.at[pos_idx.at[j]], pos_rows[b], sem_g[b])

        def wait_gather(b):
            pltpu.make_async_copy(ttab_hbm.at[tok_idx.at[0]], tok_rows[b],
                                  sem_g[b]).wait()
            pltpu.make_async_copy(ptab_hbm.at[pos_idx.at[0]], pos_rows[b],
                                  sem_g[b]).wait()

        def issue_write(j, b):
            g = (row0 + j) * C
            pltpu.async_copy(
                tok_rows[b], out_hbm.at[pl.ds(g, C), pl.ds(0, TOK_D)], sem_w[b])
            pltpu.async_copy(
                pos_rows[b], out_hbm.at[pl.ds(g, C), pl.ds(TOK_D, POS_D)],
                sem_w[b])

        def wait_write(j, b):
            g = (row0 + j) * C
            pltpu.make_async_copy(
                tok_rows[b], out_hbm.at[pl.ds(g, C), pl.ds(0, TOK_D)],
                sem_w[b]).wait()
            pltpu.make_async_copy(
                pos_rows[b], out_hbm.at[pl.ds(g, C), pl.ds(TOK_D, POS_D)],
                sem_w[b]).wait()

        DL = NBUF // 2

        def substep(j, phase, do_wait_w, do_prefetch):
            bp = (phase + DL) % NBUF
            if do_wait_w:
                wait_write(j + DL - NBUF, bp)
            if do_prefetch:
                issue_gather(j + DL, bp)
            wait_gather(phase)
            issue_write(j, phase)

        for j in range(DL):
            issue_gather(j, j % NBUF)

        for j in range(NBUF):
            substep(j, j, do_wait_w=(j + DL - NBUF >= 0), do_prefetch=True)

        @pl.loop(NBUF, S - NBUF, step=NBUF)
        def _(j0):
            for b in range(NBUF):
                substep(j0 + b, b, do_wait_w=True, do_prefetch=True)

        for b in range(NBUF):
            j = S - NBUF + b
            substep(j, b, do_wait_w=(j + DL < S), do_prefetch=(j + DL < S))

        for b in range(NBUF):
            wait_write(S - NBUF + b, (S - NBUF + b) % NBUF)

    return k(tok2, pos2, token_table, pos_table)


def kernel(tokens, pos, token_table, pos_table):
    tok2 = tokens.reshape(ROWS, C)
    pos2 = pos.reshape(ROWS, C)
    out = _sc_embed(tok2, pos2, token_table, pos_table)
    return out.reshape(B, L, TOK_D + POS_D)


# speedup vs baseline: 4.5918x; 1.1241x over previous
import functools

import jax
import jax.numpy as jnp
from jax import lax
from jax.experimental import pallas as pl
from jax.experimental.pallas import tpu as pltpu, tpu_sc as plsc

B, L = 4096, 200
TOK_D = 64
POS_D = 64
N = B * L
C = 128
NC, NS = 2, 16
NW = NC * NS
S = N // (NW * C)
ROWS = N // C
NBUF = 4


def _sc_embed(tok2, pos2, token_table, pos_table):
    mesh = plsc.VectorSubcoreMesh(core_axis_name="c", subcore_axis_name="s")

    scratch = (
        [pltpu.VMEM((S, C), jnp.int32)] * 2
        + [pltpu.VMEM((C, TOK_D), jnp.float32)] * NBUF
        + [pltpu.VMEM((C, POS_D), jnp.float32)] * NBUF
        + [pltpu.SemaphoreType.DMA] * (2 * NBUF)
        + [pltpu.VMEM_SHARED((2048, POS_D), jnp.float32)]
    )

    @functools.partial(
        pl.kernel,
        mesh=mesh,
        out_type=jax.ShapeDtypeStruct((N, TOK_D + POS_D), jnp.float32),
        compiler_params=pltpu.CompilerParams(use_tc_tiling_on_sc=False),
        scratch_types=scratch,
    )
    def k(tok_hbm, pos_hbm, ttab_hbm, ptab_hbm, out_hbm, tok_idx, pos_idx,
          *bufs):
        tok_rows = bufs[0:NBUF]
        pos_rows = bufs[NBUF:2 * NBUF]
        sem_g = bufs[2 * NBUF:3 * NBUF]
        sem_w = bufs[3 * NBUF:4 * NBUF]
        ptab_sh = bufs[4 * NBUF]

        sid = lax.axis_index("s")
        pltpu.sync_copy(ptab_hbm.at[pl.ds(sid * 128, 128)], pos_rows[0])
        pltpu.sync_copy(pos_rows[0], ptab_sh.at[pl.ds(sid * 128, 128)])
        plsc.subcore_barrier()

        wid = lax.axis_index("s") * NC + lax.axis_index("c")
        row0 = wid * S
        pltpu.sync_copy(tok_hbm.at[pl.ds(row0, S)], tok_idx)
        pltpu.sync_copy(pos_hbm.at[pl.ds(row0, S)], pos_idx)

        def issue_gather(j, b):
            pltpu.async_copy(ttab_hbm.at[tok_idx.at[j]], tok_rows[b], sem_g[b])

        def wait_gather(b, j):
            pltpu.make_async_copy(ttab_hbm.at[tok_idx.at[0]], tok_rows[b],
                                  sem_g[b]).wait()
            pltpu.sync_copy(ptab_sh.at[pos_idx.at[j]], pos_rows[b])

        def issue_write(j, b):
            g = (row0 + j) * C
            pltpu.async_copy(
                tok_rows[b], out_hbm.at[pl.ds(g, C), pl.ds(0, TOK_D)], sem_w[b])
            pltpu.async_copy(
                pos_rows[b], out_hbm.at[pl.ds(g, C), pl.ds(TOK_D, POS_D)],
                sem_w[b])

        def wait_write(j, b):
            g = (row0 + j) * C
            pltpu.make_async_copy(
                tok_rows[b], out_hbm.at[pl.ds(g, C), pl.ds(0, TOK_D)],
                sem_w[b]).wait()
            pltpu.make_async_copy(
                pos_rows[b], out_hbm.at[pl.ds(g, C), pl.ds(TOK_D, POS_D)],
                sem_w[b]).wait()

        DL = NBUF // 2

        def substep(j, phase, do_wait_w, do_prefetch):
            bp = (phase + DL) % NBUF
            if do_wait_w:
                wait_write(j + DL - NBUF, bp)
            if do_prefetch:
                issue_gather(j + DL, bp)
            wait_gather(phase, j)
            issue_write(j, phase)

        for j in range(DL):
            issue_gather(j, j % NBUF)

        for j in range(NBUF):
            substep(j, j, do_wait_w=(j + DL - NBUF >= 0), do_prefetch=True)

        @pl.loop(NBUF, S - NBUF, step=NBUF)
        def _(j0):
            for b in range(NBUF):
                substep(j0 + b, b, do_wait_w=True, do_prefetch=True)

        for b in range(NBUF):
            j = S - NBUF + b
            substep(j, b, do_wait_w=(j + DL < S), do_prefetch=(j + DL < S))

        for b in range(NBUF):
            wait_write(S - NBUF + b, (S - NBUF + b) % NBUF)

    return k(tok2, pos2, token_table, pos_table)


def kernel(tokens, pos, token_table, pos_table):
    tok2 = tokens.reshape(ROWS, C)
    pos2 = pos.reshape(ROWS, C)
    out = _sc_embed(tok2, pos2, token_table, pos_table)
    return out.reshape(B, L, TOK_D + POS_D)
